# Initial kernel scaffold; baseline (speedup 1.0000x reference)
import jax
import jax.numpy as jnp
from jax.experimental import pallas as pl


def kernel(x, edge_index, edge_attr, y, W1, b1, W2, b2, We, be):
    raise NotImplementedError("write your pallas kernel here")


# speedup vs baseline: 16.6142x; 16.6142x over previous
import functools

import jax
import jax.numpy as jnp
from jax import lax
from jax.experimental import pallas as pl
from jax.experimental.pallas import tpu as pltpu
from jax.experimental.pallas import tpu_sc as plsc

N = 10000
E = 320000
F = 128
NC, NS = 2, 16
NW = NC * NS
EPW = E // NW
CH = 128
NFULL = EPW // CH
TAIL = EPW - NFULL * CH
R_LO = 624
R_HI = N - (NS - 1) * R_LO
RCH = 104
RNCH = R_LO // RCH
RTAIL = R_HI - RNCH * RCH
H_LO = 624
H_HI = N - (NS - 1) * H_LO


def _sc_mesh():
    return plsc.VectorSubcoreMesh(core_axis_name="c", subcore_axis_name="s",
                                  num_cores=NC, num_subcores=NS)


def _sc_degree(dsts):
    @functools.partial(
        pl.kernel,
        out_type=jax.ShapeDtypeStruct((NC * N,), jnp.float32),
        mesh=_sc_mesh(),
        scratch_types=[
            pltpu.VMEM((CH,), jnp.int32),
            pltpu.VMEM((TAIL,), jnp.int32),
            pltpu.VMEM((CH,), jnp.float32),
            pltpu.VMEM((TAIL,), jnp.float32),
            pltpu.VMEM((H_HI,), jnp.float32),
            pltpu.VMEM_SHARED((N,), jnp.float32),
        ],
    )
    def k(dsts_hbm, out_hbm, didx, tdidx, ones, tones, zbuf, acc):
        cid = lax.axis_index("c")
        sid = lax.axis_index("s")
        wid = sid * NC + cid
        one16 = jnp.ones((16,), jnp.float32)
        zero16 = jnp.zeros((16,), jnp.float32)
        for j in range(CH // 16):
            ones[pl.ds(j * 16, 16)] = one16
        tones[...] = one16
        for j in range(H_HI // 16):
            zbuf[pl.ds(j * 16, 16)] = zero16

        @pl.when(sid < NS - 1)
        def _():
            pltpu.sync_copy(zbuf.at[pl.ds(0, H_LO)],
                            acc.at[pl.ds(sid * H_LO, H_LO)])

        @pl.when(sid == NS - 1)
        def _():
            pltpu.sync_copy(zbuf, acc.at[pl.ds((NS - 1) * H_LO, H_HI)])

        plsc.subcore_barrier()
        base = wid * EPW

        def chunk(c, carry):
            pltpu.sync_copy(dsts_hbm.at[pl.ds(base + c * CH, CH)], didx)
            pltpu.sync_copy(ones, acc.at[didx], add=True)
            return carry

        lax.fori_loop(0, NFULL, chunk, 0)
        pltpu.sync_copy(dsts_hbm.at[pl.ds(base + NFULL * CH, TAIL)], tdidx)
        pltpu.sync_copy(tones, acc.at[tdidx], add=True)
        plsc.subcore_barrier()

        @pl.when(sid < NS - 1)
        def _():
            pltpu.sync_copy(acc.at[pl.ds(sid * H_LO, H_LO)],
                            zbuf.at[pl.ds(0, H_LO)])
            pltpu.sync_copy(zbuf.at[pl.ds(0, H_LO)],
                            out_hbm.at[pl.ds(cid * N + sid * H_LO, H_LO)])

        @pl.when(sid == NS - 1)
        def _():
            pltpu.sync_copy(acc.at[pl.ds((NS - 1) * H_LO, H_HI)], zbuf)
            pltpu.sync_copy(
                zbuf, out_hbm.at[pl.ds(cid * N + (NS - 1) * H_LO, H_HI)])

    return k(dsts)


def _sc_scatter(g, srcs, dsts):
    @functools.partial(
        pl.kernel,
        out_type=jax.ShapeDtypeStruct((NC, N, F), jnp.float32),
        mesh=_sc_mesh(),
        scratch_types=[
            pltpu.VMEM((CH,), jnp.int32),
            pltpu.VMEM((CH,), jnp.int32),
            pltpu.VMEM((TAIL,), jnp.int32),
            pltpu.VMEM((TAIL,), jnp.int32),
            pltpu.VMEM((CH, F), jnp.float32),
            pltpu.VMEM((TAIL, F), jnp.float32),
            pltpu.VMEM((RCH, F), jnp.float32),
            pltpu.VMEM_SHARED((N, F), jnp.float32),
            pltpu.SemaphoreType.DMA,
        ],
    )
    def k(g_hbm, srcs_hbm, dsts_hbm, out_hbm,
          sidx, didx, tsidx, tdidx, rows, trows, zbuf, acc, sem):
        cid = lax.axis_index("c")
        sid = lax.axis_index("s")
        wid = sid * NC + cid
        zero16 = jnp.zeros((16,), jnp.float32)

        def zrow(r, carry):
            for j in range(F // 16):
                zbuf[r, pl.ds(j * 16, 16)] = zero16
            return carry

        lax.fori_loop(0, RCH, zrow, 0)
        row0 = sid * R_LO

        def zacc(kk, carry):
            pltpu.sync_copy(zbuf, acc.at[pl.ds(row0 + kk * RCH, RCH)])
            return carry

        lax.fori_loop(0, RNCH, zacc, 0)

        @pl.when(sid == NS - 1)
        def _():
            pltpu.sync_copy(zbuf.at[pl.ds(0, RTAIL)],
                            acc.at[pl.ds(row0 + RNCH * RCH, RTAIL)])

        plsc.subcore_barrier()
        base = wid * EPW

        def chunk(c, carry):
            off = base + c * CH
            pltpu.sync_copy(srcs_hbm.at[pl.ds(off, CH)], sidx)
            pltpu.sync_copy(dsts_hbm.at[pl.ds(off, CH)], didx)
            pltpu.async_copy(g_hbm.at[sidx], rows, sem).wait()
            pltpu.sync_copy(rows, acc.at[didx], add=True)
            return carry

        lax.fori_loop(0, NFULL, chunk, 0)
        toff = base + NFULL * CH
        pltpu.sync_copy(srcs_hbm.at[pl.ds(toff, TAIL)], tsidx)
        pltpu.sync_copy(dsts_hbm.at[pl.ds(toff, TAIL)], tdidx)
        pltpu.async_copy(g_hbm.at[tsidx], trows, sem).wait()
        pltpu.sync_copy(trows, acc.at[tdidx], add=True)
        plsc.subcore_barrier()

        def drain(kk, carry):
            r = row0 + kk * RCH
            pltpu.sync_copy(acc.at[pl.ds(r, RCH)], zbuf)
            pltpu.sync_copy(zbuf, out_hbm.at[cid, pl.ds(r, RCH)])
            return carry

        lax.fori_loop(0, RNCH, drain, 0)

        @pl.when(sid == NS - 1)
        def _():
            r = row0 + RNCH * RCH
            pltpu.sync_copy(acc.at[pl.ds(r, RTAIL)], zbuf.at[pl.ds(0, RTAIL)])
            pltpu.sync_copy(zbuf.at[pl.ds(0, RTAIL)],
                            out_hbm.at[cid, pl.ds(r, RTAIL)])

    return k(g, srcs, dsts)


BM = 1000


def _tc_pre(x, W, dinv):
    def body(x_ref, w_ref, d_ref, o_ref):
        o_ref[...] = d_ref[...] * jnp.dot(
            x_ref[...], w_ref[...], preferred_element_type=jnp.float32)

    return pl.pallas_call(
        body,
        grid=(N // BM,),
        in_specs=[pl.BlockSpec((BM, F), lambda i: (i, 0)),
                  pl.BlockSpec((F, F), lambda i: (0, 0)),
                  pl.BlockSpec((BM, 1), lambda i: (i, 0))],
        out_specs=pl.BlockSpec((BM, F), lambda i: (i, 0)),
        out_shape=jax.ShapeDtypeStruct((N, F), jnp.float32),
    )(x, W, dinv)


def _tc_mid(parts, g1, b1, W2, dinv):
    def body(p_ref, g_ref, b_ref, w_ref, d_ref, o_ref):
        d = d_ref[...]
        z = d * (p_ref[0] + p_ref[1] + g_ref[...]) + b_ref[...]
        z = jnp.maximum(z, 0.0)
        o_ref[...] = d * jnp.dot(z, w_ref[...],
                                 preferred_element_type=jnp.float32)

    return pl.pallas_call(
        body,
        grid=(N // BM,),
        in_specs=[pl.BlockSpec((NC, BM, F), lambda i: (0, i, 0)),
                  pl.BlockSpec((BM, F), lambda i: (i, 0)),
                  pl.BlockSpec((1, F), lambda i: (0, 0)),
                  pl.BlockSpec((F, F), lambda i: (0, 0)),
                  pl.BlockSpec((BM, 1), lambda i: (i, 0))],
        out_specs=pl.BlockSpec((BM, F), lambda i: (i, 0)),
        out_shape=jax.ShapeDtypeStruct((N, F), jnp.float32),
    )(parts, g1, b1.reshape(1, F), W2, dinv)


def _tc_post(parts, g2, b2, dinv):
    def body(p_ref, g_ref, b_ref, d_ref, o_ref):
        o_ref[...] = (d_ref[...] * (p_ref[0] + p_ref[1] + g_ref[...])
                      + b_ref[...])

    return pl.pallas_call(
        body,
        grid=(N // BM,),
        in_specs=[pl.BlockSpec((NC, BM, F), lambda i: (0, i, 0)),
                  pl.BlockSpec((BM, F), lambda i: (i, 0)),
                  pl.BlockSpec((1, F), lambda i: (0, 0)),
                  pl.BlockSpec((BM, 1), lambda i: (i, 0))],
        out_specs=pl.BlockSpec((BM, F), lambda i: (i, 0)),
        out_shape=jax.ShapeDtypeStruct((N, F), jnp.float32),
    )(parts, g2, b2.reshape(1, F), dinv)


def kernel(x, edge_index, edge_attr, y, W1, b1, W2, b2, We, be):
    srcs = edge_index[0]
    dsts = edge_index[1]
    degp = _sc_degree(dsts).reshape(NC, N)
    deg = degp[0] + degp[1] + 1.0
    dinv = lax.rsqrt(deg).reshape(N, 1)
    g1 = _tc_pre(x, W1, dinv)
    p1 = _sc_scatter(g1, srcs, dsts)
    g2 = _tc_mid(p1, g1, b1, W2, dinv)
    p2 = _sc_scatter(g2, srcs, dsts)
    return _tc_post(p2, g2, b2, dinv)
